# baseline (speedup 1.0000x reference)
import functools

import jax
import jax.numpy as jnp
from jax import lax
from jax.experimental import pallas as pl
from jax.experimental.pallas import tpu as pltpu
from jax.experimental.pallas import tpu_sc as plsc

_NC = 2
_NS = 16
_NW = _NC * _NS


def _sc_gather(tflat, idx3, h):
    nw, ng, g = idx3.shape
    rows_w = ng * g

    mesh = plsc.VectorSubcoreMesh(core_axis_name="c", subcore_axis_name="s")

    @functools.partial(
        pl.kernel,
        mesh=mesh,
        out_type=jax.ShapeDtypeStruct((nw * rows_w, h), jnp.float32),
        scratch_types=[
            pltpu.VMEM((ng, g), jnp.int32),
            pltpu.VMEM((2, g, h), jnp.float32),
            pltpu.SemaphoreType.DMA,
            pltpu.SemaphoreType.DMA,
        ],
        compiler_params=pltpu.CompilerParams(use_tc_tiling_on_sc=False),
    )
    def k(t_hbm, idx_hbm, out_hbm, idx_v, buf, gsem, osem):
        wid = lax.axis_index("s") * _NC + lax.axis_index("c")
        pltpu.sync_copy(idx_hbm.at[wid], idx_v)
        base = wid * rows_w

        pltpu.make_async_copy(t_hbm.at[idx_v.at[0]], buf.at[0], gsem).start()

        def step(gi, _):
            par = lax.rem(gi, 2)
            pltpu.make_async_copy(
                t_hbm.at[idx_v.at[gi]], buf.at[par], gsem
            ).wait()

            @pl.when(gi + 1 < ng)
            def _():
                @pl.when(gi >= 1)
                def _():
                    pltpu.make_async_copy(
                        buf.at[1 - par],
                        out_hbm.at[pl.ds(base + (gi - 1) * g, g)],
                        osem,
                    ).wait()

                pltpu.make_async_copy(
                    t_hbm.at[idx_v.at[gi + 1]], buf.at[1 - par], gsem
                ).start()

            pltpu.make_async_copy(
                buf.at[par], out_hbm.at[pl.ds(base + gi * g, g)], osem
            ).start()
            return 0

        lax.fori_loop(0, ng, step, 0)
        pltpu.make_async_copy(
            buf.at[lax.rem(ng - 2, 2)],
            out_hbm.at[pl.ds(base + (ng - 2) * g, g)],
            osem,
        ).wait()
        pltpu.make_async_copy(
            buf.at[lax.rem(ng - 1, 2)],
            out_hbm.at[pl.ds(base + (ng - 1) * g, g)],
            osem,
        ).wait()

    return k(tflat, idx3)


def _tc_linear(g2, wbf, b2, bblk):
    bsz, kdim = g2.shape
    h = wbf.shape[1]

    def body(g_ref, w_ref, b_ref, o_ref):
        gb = g_ref[...].astype(jnp.bfloat16)
        y = lax.dot_general(
            gb, w_ref[...],
            dimension_numbers=(((1,), (0,)), ((), ())),
            preferred_element_type=jnp.float32)
        o_ref[...] = jnp.maximum(y + b_ref[...], 0.0)

    return pl.pallas_call(
        body,
        grid=(bsz // bblk,),
        in_specs=[
            pl.BlockSpec((bblk, kdim), lambda i: (i, 0)),
            pl.BlockSpec((kdim, h), lambda i: (0, 0)),
            pl.BlockSpec((1, h), lambda i: (0, 0)),
        ],
        out_specs=pl.BlockSpec((bblk, h), lambda i: (i, 0)),
        out_shape=jax.ShapeDtypeStruct((bsz, h), jnp.float32),
        compiler_params=pltpu.CompilerParams(
            dimension_semantics=("parallel",)),
    )(g2, wbf, b2)


def kernel(x, tables, W, b):
    bsz, nf = x.shape
    _, vocab, h = tables.shape

    tflat = tables.reshape(nf * vocab, h)

    f_off = (jnp.arange(nf, dtype=jnp.int32) * vocab)[None, :]
    idx = (x.astype(jnp.int32) + f_off).reshape(-1)
    g = 4 * nf
    ng = (bsz * nf) // (_NW * g)
    idx3 = idx.reshape(_NW, ng, g)

    gat = _sc_gather(tflat, idx3, h)
    g2 = gat.reshape(bsz, nf * h)

    wbf = W.astype(jnp.bfloat16)
    b2 = b.reshape(1, h)
    return _tc_linear(g2, wbf, b2, 1024)
